# baseline (speedup 1.0000x reference)
import functools

import jax
import jax.numpy as jnp
import numpy as np
from jax import lax
from jax.experimental import pallas as pl
from jax.experimental.pallas import tpu as pltpu
from jax.experimental.pallas import tpu_sc as plsc

N = 10000
E = 160000
E_SEQ = 60000
C_S = 128
C_V = 16
C_Z = 128
H = 8
C_H = 16
P_QK = 4
P_V = 8
H_TIME = 64
SCALAR_H = 128
N_LAYERS = 2

N_PAD = 10240

_NC, _NS = 2, 16
_NW = _NC * _NS
_NROW = 10240
_FH = 128

E_P = 163840
ES_P = 65536


@functools.lru_cache(maxsize=None)
def _make_gather(V, D, B, chunk):
    per_w = B // _NW
    n_chunks = per_w // chunk
    R = 2 if chunk * D * 4 * 3 > 460_000 else 3
    b2_shape = (chunk, D) if R == 3 else (8, 128)
    mesh = plsc.VectorSubcoreMesh(core_axis_name="c", subcore_axis_name="s")

    @functools.partial(
        pl.kernel, mesh=mesh,
        out_type=jax.ShapeDtypeStruct((B, D), jnp.float32),
        scratch_types=[
            pltpu.VMEM((n_chunks, chunk), jnp.int32),
            pltpu.VMEM((chunk, D), jnp.float32),
            pltpu.VMEM((chunk, D), jnp.float32),
            pltpu.VMEM(b2_shape, jnp.float32),
            pltpu.SemaphoreType.DMA,
            pltpu.SemaphoreType.DMA,
            pltpu.SemaphoreType.DMA,
            pltpu.SemaphoreType.DMA,
            pltpu.SemaphoreType.DMA,
            pltpu.SemaphoreType.DMA,
        ],
    )
    def gath(table_hbm, idx_hbm, out_hbm, idx_v,
             buf0, buf1, buf2, g0, g1, g2, s0, s1, s2):
        wid = lax.axis_index("s") * _NC + lax.axis_index("c")
        row0 = wid * n_chunks
        pltpu.sync_copy(idx_hbm.at[wid], idx_v)
        bufs = (buf0, buf1, buf2)[:R]
        gs = (g0, g1, g2)[:R]
        ss = (s0, s1, s2)[:R]

        def fire(j, b):
            pltpu.async_copy(table_hbm.at[idx_v.at[j]], bufs[b], gs[b])

        def wait_g(b):
            pltpu.make_async_copy(table_hbm.at[idx_v.at[0]], bufs[b], gs[b]).wait()

        def fire_st(j, b):
            pltpu.async_copy(bufs[b], out_hbm.at[pl.ds((row0 + j) * chunk, chunk)],
                             ss[b])

        def wait_st(b):
            pltpu.make_async_copy(bufs[b], out_hbm.at[pl.ds(0, chunk)], ss[b]).wait()

        for b in range(min(R, n_chunks)):
            fire(b, b)

        def ring(g, carry):
            j0 = g * R
            for b in range(R):
                wait_g(b)
                fire_st(j0 + b, b)
            for b in range(R):
                wait_st(b)

                @pl.when(j0 + b + R < n_chunks)
                def _():
                    fire(j0 + b + R, b)
            return carry

        lax.fori_loop(0, n_chunks // R, ring, 0)
        for j in range(n_chunks - n_chunks % R, n_chunks):
            b = j % R
            wait_g(b)
            fire_st(j, b)
            wait_st(b)

    return gath


def _sc_gather(table, idx_pad):
    B = idx_pad.shape[0]
    D = table.shape[1]
    chunk = 128
    fn = _make_gather(table.shape[0], D, B, chunk)
    return fn(table, idx_pad.reshape(_NW, B // (_NW * chunk), chunk))


@functools.lru_cache(maxsize=None)
def _make_scatter(B):
    CH = 128
    n_chunks = B // (_NW * CH)
    rows_t = _NROW // _NS
    mesh = plsc.VectorSubcoreMesh(core_axis_name="c", subcore_axis_name="s")

    @functools.partial(
        pl.kernel, mesh=mesh,
        out_type=jax.ShapeDtypeStruct((6 * _NROW, _FH), jnp.float32),
        scratch_types=[
            pltpu.VMEM((n_chunks, CH), jnp.int32),
            pltpu.VMEM((CH, _FH), jnp.float32),
            pltpu.VMEM((CH, _FH), jnp.float32),
            pltpu.VMEM_SHARED((_NROW, _FH), jnp.float32),
            pltpu.SemaphoreType.DMA,
            pltpu.SemaphoreType.DMA,
        ],
    )
    def scat(vals_hbm, idx_hbm, zeros_hbm, out_hbm, idx_v, vb0, vb1, acc, l0, l1):
        c = lax.axis_index("c")
        s = lax.axis_index("s")
        wid = s * _NC + c
        r0 = s * rows_t
        row0 = wid * n_chunks
        pltpu.sync_copy(idx_hbm.at[wid], idx_v)
        vbufs, ls = (vb0, vb1), (l0, l1)

        for g in range(3):
            col = g * _FH

            def fire_ld(j, b):
                pltpu.async_copy(
                    vals_hbm.at[pl.ds((row0 + j) * CH, CH), pl.ds(col, _FH)],
                    vbufs[b], ls[b])

            def wait_ld(b):
                pltpu.make_async_copy(
                    vals_hbm.at[pl.ds(0, CH), pl.ds(col, _FH)],
                    vbufs[b], ls[b]).wait()

            pltpu.sync_copy(zeros_hbm, acc.at[pl.ds(r0, rows_t)])
            plsc.subcore_barrier()
            fire_ld(0, 0)
            if n_chunks > 1:
                fire_ld(1, 1)

            def pair(gg, carry):
                j0 = 2 * gg
                for b in range(2):
                    j = j0 + b
                    wait_ld(b)
                    pltpu.sync_copy(vbufs[b], acc.at[idx_v.at[j]], add=True)

                    @pl.when(j + 2 < n_chunks)
                    def _():
                        fire_ld(j + 2, b)
                return carry

            lax.fori_loop(0, n_chunks // 2, pair, 0)
            if n_chunks % 2:
                b = (n_chunks - 1) % 2
                wait_ld(b)
                pltpu.sync_copy(vbufs[b], acc.at[idx_v.at[n_chunks - 1]], add=True)
            plsc.subcore_barrier()
            pltpu.sync_copy(acc.at[pl.ds(r0, rows_t)],
                            out_hbm.at[pl.ds((2 * g) * _NROW + c * _NROW + r0,
                                             rows_t)])
            plsc.subcore_barrier()

    return scat


def _sc_scatter3(vals, idx_pad):
    B = idx_pad.shape[0]
    fn = _make_scatter(B)
    zeros = jnp.zeros((_NROW // _NS, _FH), jnp.float32)
    out = fn(vals, idx_pad.reshape(_NW, B // (_NW * 128), 128), zeros)
    return [out[2 * g * _NROW:2 * g * _NROW + N]
            + out[(2 * g + 1) * _NROW:(2 * g + 1) * _NROW + N] for g in range(3)]


def _rup(n, m):
    return ((n + m - 1) // m) * m


@functools.lru_cache(maxsize=None)
def _make_mm(B, K, M, act):
    def body(x_ref, w_ref, o_ref):
        y = jnp.dot(x_ref[...], w_ref[...], preferred_element_type=jnp.float32)
        if act == 'relu':
            y = jnp.maximum(y, 0.0)
        elif act == 'sigmoid':
            y = jax.nn.sigmoid(y)
        o_ref[...] = y

    return pl.pallas_call(
        body,
        grid=(B // 512,),
        in_specs=[
            pl.BlockSpec((512, K), lambda i: (i, 0)),
            pl.BlockSpec((K, M), lambda i: (0, 0)),
        ],
        out_specs=pl.BlockSpec((512, M), lambda i: (i, 0)),
        out_shape=jax.ShapeDtypeStruct((B, M), jnp.float32),
    )


def _mm(x, w, act=None):
    B0, K0 = x.shape
    M0 = w.shape[1]
    B, K, M = _rup(B0, 512), _rup(K0, 128), _rup(M0, 128)
    xp = x if (B0 == B and K0 == K) else jnp.zeros((B, K), jnp.float32).at[:B0, :K0].set(x)
    wp = w if (K0 == K and M0 == M) else jnp.zeros((K, M), jnp.float32).at[:K0, :M0].set(w)
    y = _make_mm(B, K, M, act)(xp, wp)
    return y[:B0, :M0]


def _edge_mlp_body(gs_ref, gd_ref, z_ref, w1s_ref, w1d_ref, w1z_ref, w2_ref, o_ref):
    h = jnp.dot(gs_ref[...], w1s_ref[...], preferred_element_type=jnp.float32)
    h += jnp.dot(gd_ref[...], w1d_ref[...], preferred_element_type=jnp.float32)
    h += jnp.dot(z_ref[...], w1z_ref[...], preferred_element_type=jnp.float32)
    h = jnp.maximum(h, 0.0)
    y = jnp.dot(h, w2_ref[...], preferred_element_type=jnp.float32)
    m = jnp.mean(y, axis=1, keepdims=True)
    v = jnp.mean((y - m) ** 2, axis=1, keepdims=True)
    o_ref[...] = (y - m) / jnp.sqrt(v + 1e-5)


@functools.lru_cache(maxsize=None)
def _make_edge_mlp(B):
    wspec = pl.BlockSpec((C_S, C_S), lambda i: (0, 0))
    xspec = pl.BlockSpec((512, C_S), lambda i: (i, 0))
    return pl.pallas_call(
        _edge_mlp_body,
        grid=(B // 512,),
        in_specs=[xspec, xspec, xspec, wspec, wspec, wspec, wspec],
        out_specs=xspec,
        out_shape=jax.ShapeDtypeStruct((B, C_S), jnp.float32),
    )


def _edge_tr_sc(p, z, gsrc, gdst):
    B = z.shape[0]
    w1 = p['W1']
    return _make_edge_mlp(B)(gsrc, gdst, z, w1[:C_S], w1[C_S:2 * C_S],
                             w1[2 * C_S:], p['W2'])


def _logits_body(gd_ref, ga_ref, z_ref, s1_ref, s2g_ref, wb_ref, o_ref):
    gd = gd_ref[...]
    ga = ga_ref[...]
    qk = jnp.dot(gd[:, :C_S] * ga[:, :C_S], s1_ref[...],
                 preferred_element_type=jnp.float32)
    diff = gd[:, C_S:224] - ga[:, C_S:224]
    d2g = jnp.dot(diff * diff, s2g_ref[...], preferred_element_type=jnp.float32)
    zb = jnp.dot(z_ref[...], wb_ref[...], preferred_element_type=jnp.float32)
    o_ref[...] = (qk / np.sqrt(C_H) + zb) / np.sqrt(3.0) - 0.5 * d2g


@functools.lru_cache(maxsize=None)
def _make_logits(B):
    return pl.pallas_call(
        _logits_body,
        grid=(B // 512,),
        in_specs=[
            pl.BlockSpec((512, 256), lambda i: (i, 0)),
            pl.BlockSpec((512, 256), lambda i: (i, 0)),
            pl.BlockSpec((512, C_Z), lambda i: (i, 0)),
            pl.BlockSpec((C_S, H), lambda i: (0, 0)),
            pl.BlockSpec((96, H), lambda i: (0, 0)),
            pl.BlockSpec((C_Z, H), lambda i: (0, 0)),
        ],
        out_specs=pl.BlockSpec((512, H), lambda i: (i, 0)),
        out_shape=jax.ShapeDtypeStruct((B, H), jnp.float32),
    )


def _values_body(e_real, la_ref, gb_ref, r16_ref, s3_ref, p1_ref, p2_ref, o_ref):
    rowid = pl.program_id(0) * 512 + lax.broadcasted_iota(jnp.int32, (512, 1), 0)
    mask = (rowid < e_real).astype(jnp.float32)
    a = jnp.exp(la_ref[...]) * mask
    gb = gb_ref[...]
    o_ref[:, :128] = jnp.dot(a, r16_ref[...], preferred_element_type=jnp.float32) * gb[:, :C_S]
    avp = jnp.dot(a, s3_ref[...], preferred_element_type=jnp.float32) * gb[:, C_S:320]
    o_ref[:, 128:256] = avp[:, :128]
    o_ref[:, 256:384] = (jnp.dot(avp[:, 128:], p1_ref[...], preferred_element_type=jnp.float32)
                         + jnp.dot(a, p2_ref[...], preferred_element_type=jnp.float32))


@functools.lru_cache(maxsize=None)
def _make_values(B, e_real):
    return pl.pallas_call(
        functools.partial(_values_body, e_real),
        grid=(B // 512,),
        in_specs=[
            pl.BlockSpec((512, H), lambda i: (i, 0)),
            pl.BlockSpec((512, 384), lambda i: (i, 0)),
            pl.BlockSpec((H, 128), lambda i: (0, 0)),
            pl.BlockSpec((H, 192), lambda i: (0, 0)),
            pl.BlockSpec((64, 128), lambda i: (0, 0)),
            pl.BlockSpec((H, 128), lambda i: (0, 0)),
        ],
        out_specs=pl.BlockSpec((512, 384), lambda i: (i, 0)),
        out_shape=jax.ShapeDtypeStruct((B, 384), jnp.float32),
    )


_NPC = {}


def _np_consts():
    if 'S1' not in _NPC:
        S1 = np.zeros((C_S, H), np.float32)
        for h in range(H):
            S1[h * C_H:(h + 1) * C_H, h] = 1.0
        S2 = np.zeros((96, H), np.float32)
        for h in range(H):
            S2[h * 12:(h + 1) * 12, h] = 1.0
        R16 = np.zeros((H, 128), np.float32)
        for h in range(H):
            R16[h, h * C_H:(h + 1) * C_H] = 1.0
        S3 = np.zeros((H, 192), np.float32)
        for h in range(H):
            S3[h, h * 24:(h + 1) * 24] = 1.0
        P1 = np.zeros((64, 128), np.float32)
        P1[:64, :64] = np.eye(64, dtype=np.float32)
        P2 = np.zeros((H, 128), np.float32)
        for h in range(H):
            P2[h, 64 + h] = 1.0
        _NPC.update(S1=jnp.asarray(S1), S2=jnp.asarray(S2), R16=jnp.asarray(R16),
                    S3=jnp.asarray(S3), P1=jnp.asarray(P1), P2=jnp.asarray(P2))
    return _NPC


def _nt_body(x_ref, w1_ref, w2_ref, o_ref):
    h = jnp.maximum(jnp.dot(x_ref[...], w1_ref[...],
                            preferred_element_type=jnp.float32), 0.0)
    o_ref[...] = jnp.dot(h, w2_ref[...], preferred_element_type=jnp.float32)


def _nt_pallas(x, w1, w2):
    xp = jnp.zeros((N_PAD, C_S), jnp.float32).at[:N].set(x)
    out = pl.pallas_call(
        _nt_body,
        grid=(N_PAD // 512,),
        in_specs=[
            pl.BlockSpec((512, C_S), lambda i: (i, 0)),
            pl.BlockSpec((C_S, 2 * C_S), lambda i: (0, 0)),
            pl.BlockSpec((2 * C_S, C_S), lambda i: (0, 0)),
        ],
        out_specs=pl.BlockSpec((512, C_S), lambda i: (i, 0)),
        out_shape=jax.ShapeDtypeStruct((N_PAD, C_S), jnp.float32),
    )(xp, w1, w2)
    return out[:N]


def _quat_to_rot(q):
    q = q / (jnp.linalg.norm(q, axis=-1, keepdims=True) + 1e-8)
    w, x, y, z = q[..., 0], q[..., 1], q[..., 2], q[..., 3]
    R = jnp.stack([1 - 2 * (y * y + z * z), 2 * (x * y - w * z), 2 * (x * z + w * y),
                   2 * (x * y + w * z), 1 - 2 * (x * x + z * z), 2 * (y * z - w * x),
                   2 * (x * z - w * y), 2 * (y * z + w * x), 1 - 2 * (x * x + y * y)], axis=-1)
    return R.reshape(q.shape[:-1] + (3, 3))


def _ln(x):
    m = x.mean(-1, keepdims=True)
    v = ((x - m) ** 2).mean(-1, keepdims=True)
    return (x - m) / jnp.sqrt(v + 1e-5)


def _vln(v):
    n2 = (v ** 2).sum(-1).mean(-1, keepdims=True)
    return v / jnp.sqrt(n2 + 1e-5)[..., None]


def _rbf(d, n, d_max=20.0):
    mu = jnp.linspace(0.0, d_max, n)
    sigma = d_max / n
    return jnp.exp(-(((d[..., None] - mu) / sigma) ** 2))


def _posemb(diff, n):
    freq = jnp.exp(jnp.arange(0, n, 2, dtype=jnp.float32) * (-np.log(10000.0) / n))
    ang = diff[..., None].astype(jnp.float32) * freq
    return jnp.concatenate([jnp.cos(ang), jnp.sin(ang)], axis=-1)


def _pad_idx(idx, B):
    return jnp.concatenate([idx, jnp.zeros((B - idx.shape[0],), jnp.int32)])


def _attn_sc(p, s, rots, trans, z, src_pad, dst_pad, e_real, v_in):
    cst = _np_consts()
    vn = jnp.sqrt((v_in ** 2).sum(-1) + 1e-8)
    wcat = jnp.concatenate([p['Wq'], p['Wk'], p['Wv'], p['Wqp'], p['Wkp'], p['Wvp']], axis=1)
    x2 = jnp.concatenate([s, vn], axis=1)
    w2 = jnp.concatenate([wcat, p['Wvn'] @ wcat], axis=0)
    proj = _mm(x2, w2)
    q, k, v = proj[:, :C_S], proj[:, C_S:256], proj[:, 256:384]

    def to_global(flat, npts):
        pts = flat.reshape(N, H, npts, 3)
        return (jnp.einsum('nij,nhpj->nhpi', rots, pts)
                + trans[:, None, None, :]).reshape(N, H * npts * 3)

    qp = to_global(proj[:, 384:480], P_QK)
    kp = to_global(proj[:, 480:576], P_QK)
    vp = to_global(proj[:, 576:768], P_V)

    pad32 = jnp.zeros((N, 32), jnp.float32)
    dst_tab = jnp.concatenate([q, qp, pad32], axis=1)
    ga_tab = jnp.concatenate([k, kp, pad32], axis=1)
    gb_tab = jnp.concatenate([v, vp, pad32, pad32], axis=1)
    gd = _sc_gather(dst_tab, dst_pad)
    ga = _sc_gather(ga_tab, src_pad)
    gb = _sc_gather(gb_tab, src_pad)

    Bp = src_pad.shape[0]
    gamma = jax.nn.softplus(p['head_w'])
    s2g = cst['S2'] * gamma[None, :]
    logits = _make_logits(Bp)(gd, ga, z, cst['S1'], s2g, p['Wb'])
    m = jnp.max(logits)
    vals = _make_values(Bp, e_real)(logits - m, gb, cst['R16'], cst['S3'],
                                    cst['P1'], cst['P2'])
    s0, s1, s2 = _sc_scatter3(vals, dst_pad)

    den = s2[:, 64:72]
    deninv = 1.0 / jnp.maximum(den, 1e-30)
    o = s0.reshape(N, H, C_H) * deninv[:, :, None]
    op = (jnp.concatenate([s1, s2[:, :64]], axis=1).reshape(N, H, P_V * 3)
          * deninv[:, :, None]).reshape(N, H, P_V, 3)
    op_local = jnp.einsum('nji,nhpj->nhpi', rots, op - trans[:, None, None, :])
    opn = jnp.sqrt((op_local ** 2).sum(-1) + 1e-8)
    feat = jnp.concatenate([o.reshape(N, -1), op_local.reshape(N, -1),
                            opn.reshape(N, -1)], axis=-1)
    ds = _mm(feat, p['Wo_s'])
    dv = _mm(op_local.reshape(N, H * P_V * 3),
             jnp.kron(p['Wo_v'], jnp.eye(3, dtype=jnp.float32)))
    return ds, dv.reshape(N, C_V, 3)


def kernel(quats, trans, t, noising_mask, x_mask, edge_index, seq_edge_index, params):
    src = _pad_idx(edge_index[0], E_P)
    dst = _pad_idx(edge_index[1], E_P)
    ssrc = _pad_idx(seq_edge_index[0], ES_P)
    sdst = _pad_idx(seq_edge_index[1], ES_P)
    all_idx = jnp.concatenate([src, dst, ssrc, sdst])

    rots = _quat_to_rot(quats)
    keep = (~x_mask).astype(jnp.float32)
    center = trans.mean(axis=0, keepdims=True)
    trans = (trans - center) * 0.1

    trans128 = jnp.zeros((N, 128), jnp.float32).at[:, :3].set(trans)
    gt = _sc_gather(trans128, all_idx)[:, :3]
    gt_src, gt_dst = gt[:E_P], gt[E_P:2 * E_P]
    gt_ssrc, gt_sdst = gt[2 * E_P:2 * E_P + ES_P], gt[2 * E_P + ES_P:]

    def efeats(gs, gdd, spad, dpad):
        d = jnp.sqrt(((gs - gdd) ** 2).sum(-1) + 1e-12)
        return jnp.concatenate([_rbf(d, C_Z // 2), _posemb(spad - dpad, C_Z // 2)],
                               axis=-1)

    z = efeats(gt_src, gt_dst, src, dst)
    zs = efeats(gt_ssrc, gt_sdst, ssrc, sdst)

    ft = _rbf(t, H_TIME, 1.0)
    et = jax.nn.relu(_mm(jax.nn.relu(_mm(ft, params['Wt1']) + params['bt1']),
                         params['Wt2']) + params['bt2'])
    res_pos = _posemb(jnp.arange(N), C_S)
    node = _mm(jnp.concatenate([res_pos, et, noising_mask[:, None]], axis=-1),
               params['Wemb']) + params['bemb']
    vecs = jnp.zeros((N, C_V, 3), jnp.float32)

    for lp in params['layers']:
        gn = _sc_gather(node, all_idx)
        z = _edge_tr_sc(lp['edge_tr'], z, gn[:E_P], gn[E_P:2 * E_P])
        zs = _edge_tr_sc(lp['seq_edge_tr'], zs, gn[2 * E_P:2 * E_P + ES_P],
                         gn[2 * E_P + ES_P:])
        ds, dv = _attn_sc(lp['attn_seq'], node, rots, trans, zs, ssrc, sdst,
                          E_SEQ, vecs)
        node = _ln(node + ds * keep[:, None])
        vecs = _vln(vecs + dv * keep[:, None, None])
        ds, dv = _attn_sc(lp['attn_spatial'], node, rots, trans, z, src, dst,
                          E, vecs)
        node = _ln(node + ds * keep[:, None])
        vecs = _vln(vecs + dv * keep[:, None, None])
        vn = jnp.sqrt((vecs ** 2).sum(-1) + 1e-8)
        h = _mm(jnp.concatenate([node, vn], axis=-1), lp['lfu_W1'], act='relu')
        ds = _mm(h, lp['lfu_W2'])
        gate = _mm(node, lp['lfu_Wg'], act='sigmoid')
        dv = gate[..., None] * _mm(vecs.reshape(N, 3 * C_V),
                                   jnp.kron(lp['lfu_Wvm'],
                                            jnp.eye(3, dtype=jnp.float32))
                                   ).reshape(N, C_V, 3)
        node = _ln(node + ds * keep[:, None])
        vecs = _vln(vecs + dv * keep[:, None, None])
        node = _ln(node + _nt_pallas(node, lp['nt_W1'], lp['nt_W2']))
        vecs = vecs * jax.nn.sigmoid(_mm(node, lp['nt_Wg']))[..., None]
        node = node * keep[:, None]
        vecs = vecs * keep[:, None, None]
        u = (_mm(node * noising_mask[:, None], lp['bb_Ws'])
             + _mm((vecs * noising_mask[:, None, None]).reshape(N, -1), lp['bb_Wv']))
        u = u * noising_mask[:, None]
        new_trans = trans + jnp.einsum('nij,nj->ni', rots, u[:, 3:])
        Ru = _quat_to_rot(jnp.concatenate([jnp.ones((N, 1), jnp.float32), u[:, :3]], axis=-1))
        rots = jnp.einsum('nij,njk->nik', rots, Ru)
        trans = new_trans
    trans = trans * 10.0 + center
    return node, trans, rots, vecs


# speedup vs baseline: 1.1463x; 1.1463x over previous
import functools

import jax
import jax.numpy as jnp
import numpy as np
from jax import lax
from jax.experimental import pallas as pl
from jax.experimental.pallas import tpu as pltpu
from jax.experimental.pallas import tpu_sc as plsc

N = 10000
E = 160000
E_SEQ = 60000
C_S = 128
C_V = 16
C_Z = 128
H = 8
C_H = 16
P_QK = 4
P_V = 8
H_TIME = 64
SCALAR_H = 128
N_LAYERS = 2

N_PAD = 10240

_NC, _NS = 2, 16
_NW = _NC * _NS
_NROW = 10240
_FH = 128

E_P = 163840
ES_P = 61440


@functools.lru_cache(maxsize=None)
def _make_gather(V, D, B, chunk):
    per_w = B // _NW
    n_chunks = per_w // chunk
    R = 2 if chunk * D * 4 * 3 > 460_000 else 3
    b2_shape = (chunk, D) if R == 3 else (8, 128)
    mesh = plsc.VectorSubcoreMesh(core_axis_name="c", subcore_axis_name="s")

    @functools.partial(
        pl.kernel, mesh=mesh,
        out_type=jax.ShapeDtypeStruct((B, D), jnp.float32),
        scratch_types=[
            pltpu.VMEM((n_chunks, chunk), jnp.int32),
            pltpu.VMEM((chunk, D), jnp.float32),
            pltpu.VMEM((chunk, D), jnp.float32),
            pltpu.VMEM(b2_shape, jnp.float32),
            pltpu.SemaphoreType.DMA,
            pltpu.SemaphoreType.DMA,
            pltpu.SemaphoreType.DMA,
            pltpu.SemaphoreType.DMA,
            pltpu.SemaphoreType.DMA,
            pltpu.SemaphoreType.DMA,
        ],
    )
    def gath(table_hbm, idx_hbm, out_hbm, idx_v,
             buf0, buf1, buf2, g0, g1, g2, s0, s1, s2):
        wid = lax.axis_index("s") * _NC + lax.axis_index("c")
        row0 = wid * n_chunks
        pltpu.sync_copy(idx_hbm.at[wid], idx_v)
        bufs = (buf0, buf1, buf2)[:R]
        gs = (g0, g1, g2)[:R]
        ss = (s0, s1, s2)[:R]

        def fire(j, b):
            pltpu.async_copy(table_hbm.at[idx_v.at[j]], bufs[b], gs[b])

        def wait_g(b):
            pltpu.make_async_copy(table_hbm.at[idx_v.at[0]], bufs[b], gs[b]).wait()

        def fire_st(j, b):
            pltpu.async_copy(bufs[b], out_hbm.at[pl.ds((row0 + j) * chunk, chunk)],
                             ss[b])

        def wait_st(b):
            pltpu.make_async_copy(bufs[b], out_hbm.at[pl.ds(0, chunk)], ss[b]).wait()

        for b in range(min(R, n_chunks)):
            fire(b, b)

        def ring(g, carry):
            j0 = g * R
            for b in range(R):
                wait_g(b)
                fire_st(j0 + b, b)
            for b in range(R):
                wait_st(b)

                @pl.when(j0 + b + R < n_chunks)
                def _():
                    fire(j0 + b + R, b)
            return carry

        lax.fori_loop(0, n_chunks // R, ring, 0)
        for j in range(n_chunks - n_chunks % R, n_chunks):
            b = j % R
            wait_g(b)
            fire_st(j, b)
            wait_st(b)

    return gath


def _sc_gather(table, idx_pad):
    B = idx_pad.shape[0]
    D = table.shape[1]
    chunk = 128 if D <= 256 else 64
    fn = _make_gather(table.shape[0], D, B, chunk)
    return fn(table, idx_pad.reshape(_NW, B // (_NW * chunk), chunk))


@functools.lru_cache(maxsize=None)
def _make_scatter(B):
    CH = 128
    n_chunks = B // (_NW * CH)
    rows_t = _NROW // _NS
    mesh = plsc.VectorSubcoreMesh(core_axis_name="c", subcore_axis_name="s")

    @functools.partial(
        pl.kernel, mesh=mesh,
        out_type=jax.ShapeDtypeStruct((6 * _NROW, _FH), jnp.float32),
        scratch_types=[
            pltpu.VMEM((n_chunks, CH), jnp.int32),
            pltpu.VMEM((CH, _FH), jnp.float32),
            pltpu.VMEM((CH, _FH), jnp.float32),
            pltpu.VMEM_SHARED((_NROW, _FH), jnp.float32),
            pltpu.SemaphoreType.DMA,
            pltpu.SemaphoreType.DMA,
        ],
    )
    def scat(vals_hbm, idx_hbm, zeros_hbm, out_hbm, idx_v, vb0, vb1, acc, l0, l1):
        c = lax.axis_index("c")
        s = lax.axis_index("s")
        wid = s * _NC + c
        r0 = s * rows_t
        row0 = wid * n_chunks
        pltpu.sync_copy(idx_hbm.at[wid], idx_v)
        vbufs, ls = (vb0, vb1), (l0, l1)

        for g in range(3):
            col = g * _FH

            def fire_ld(j, b):
                pltpu.async_copy(
                    vals_hbm.at[pl.ds((row0 + j) * CH, CH), pl.ds(col, _FH)],
                    vbufs[b], ls[b])

            def wait_ld(b):
                pltpu.make_async_copy(
                    vals_hbm.at[pl.ds(0, CH), pl.ds(col, _FH)],
                    vbufs[b], ls[b]).wait()

            pltpu.sync_copy(zeros_hbm, acc.at[pl.ds(r0, rows_t)])
            plsc.subcore_barrier()
            fire_ld(0, 0)
            if n_chunks > 1:
                fire_ld(1, 1)

            def pair(gg, carry):
                j0 = 2 * gg
                for b in range(2):
                    j = j0 + b
                    wait_ld(b)
                    pltpu.sync_copy(vbufs[b], acc.at[idx_v.at[j]], add=True)

                    @pl.when(j + 2 < n_chunks)
                    def _():
                        fire_ld(j + 2, b)
                return carry

            lax.fori_loop(0, n_chunks // 2, pair, 0)
            if n_chunks % 2:
                b = (n_chunks - 1) % 2
                wait_ld(b)
                pltpu.sync_copy(vbufs[b], acc.at[idx_v.at[n_chunks - 1]], add=True)
            plsc.subcore_barrier()
            pltpu.sync_copy(acc.at[pl.ds(r0, rows_t)],
                            out_hbm.at[pl.ds((2 * g) * _NROW + c * _NROW + r0,
                                             rows_t)])
            plsc.subcore_barrier()

    return scat


def _sc_scatter3(vals, idx_pad):
    B = idx_pad.shape[0]
    fn = _make_scatter(B)
    zeros = jnp.zeros((_NROW // _NS, _FH), jnp.float32)
    out = fn(vals, idx_pad.reshape(_NW, B // (_NW * 128), 128), zeros)
    return [out[2 * g * _NROW:2 * g * _NROW + N]
            + out[(2 * g + 1) * _NROW:(2 * g + 1) * _NROW + N] for g in range(3)]


def _rup(n, m):
    return ((n + m - 1) // m) * m


@functools.lru_cache(maxsize=None)
def _make_mm(B, K, M, act):
    def body(x_ref, w_ref, o_ref):
        y = jnp.dot(x_ref[...], w_ref[...], preferred_element_type=jnp.float32)
        if act == 'relu':
            y = jnp.maximum(y, 0.0)
        elif act == 'sigmoid':
            y = jax.nn.sigmoid(y)
        o_ref[...] = y

    return pl.pallas_call(
        body,
        grid=(B // 512,),
        in_specs=[
            pl.BlockSpec((512, K), lambda i: (i, 0)),
            pl.BlockSpec((K, M), lambda i: (0, 0)),
        ],
        out_specs=pl.BlockSpec((512, M), lambda i: (i, 0)),
        out_shape=jax.ShapeDtypeStruct((B, M), jnp.float32),
    )


def _mm(x, w, act=None):
    B0, K0 = x.shape
    M0 = w.shape[1]
    B, K, M = _rup(B0, 512), _rup(K0, 128), _rup(M0, 128)
    xp = x if (B0 == B and K0 == K) else jnp.zeros((B, K), jnp.float32).at[:B0, :K0].set(x)
    wp = w if (K0 == K and M0 == M) else jnp.zeros((K, M), jnp.float32).at[:K0, :M0].set(w)
    y = _make_mm(B, K, M, act)(xp, wp)
    return y[:B0, :M0]


def _edge_mlp_body(gs_ref, gd_ref, z_ref, w1s_ref, w1d_ref, w1z_ref, w2_ref, o_ref):
    h = jnp.dot(gs_ref[...], w1s_ref[...], preferred_element_type=jnp.float32)
    h += jnp.dot(gd_ref[...], w1d_ref[...], preferred_element_type=jnp.float32)
    h += jnp.dot(z_ref[...], w1z_ref[...], preferred_element_type=jnp.float32)
    h = jnp.maximum(h, 0.0)
    y = jnp.dot(h, w2_ref[...], preferred_element_type=jnp.float32)
    m = jnp.mean(y, axis=1, keepdims=True)
    v = jnp.mean((y - m) ** 2, axis=1, keepdims=True)
    o_ref[...] = (y - m) / jnp.sqrt(v + 1e-5)


@functools.lru_cache(maxsize=None)
def _make_edge_mlp(B):
    wspec = pl.BlockSpec((C_S, C_S), lambda i: (0, 0))
    xspec = pl.BlockSpec((512, C_S), lambda i: (i, 0))
    return pl.pallas_call(
        _edge_mlp_body,
        grid=(B // 512,),
        in_specs=[xspec, xspec, xspec, wspec, wspec, wspec, wspec],
        out_specs=xspec,
        out_shape=jax.ShapeDtypeStruct((B, C_S), jnp.float32),
    )


def _edge_tr_sc(p, z, gsrc, gdst):
    B = z.shape[0]
    w1 = p['W1']
    return _make_edge_mlp(B)(gsrc, gdst, z, w1[:C_S], w1[C_S:2 * C_S],
                             w1[2 * C_S:], p['W2'])


def _logits_body(gd_ref, ga_ref, z_ref, s1_ref, s2g_ref, wb_ref, o_ref):
    gd = gd_ref[...]
    ga = ga_ref[...]
    qk = jnp.dot(gd[:, :C_S] * ga[:, :C_S], s1_ref[...],
                 preferred_element_type=jnp.float32)
    diff = gd[:, C_S:224] - ga[:, C_S:224]
    d2g = jnp.dot(diff * diff, s2g_ref[...], preferred_element_type=jnp.float32)
    zb = jnp.dot(z_ref[...], wb_ref[...], preferred_element_type=jnp.float32)
    o_ref[...] = (qk / np.sqrt(C_H) + zb) / np.sqrt(3.0) - 0.5 * d2g


@functools.lru_cache(maxsize=None)
def _make_logits(B):
    return pl.pallas_call(
        _logits_body,
        grid=(B // 512,),
        in_specs=[
            pl.BlockSpec((512, 256), lambda i: (i, 0)),
            pl.BlockSpec((512, 256), lambda i: (i, 0)),
            pl.BlockSpec((512, C_Z), lambda i: (i, 0)),
            pl.BlockSpec((C_S, H), lambda i: (0, 0)),
            pl.BlockSpec((96, H), lambda i: (0, 0)),
            pl.BlockSpec((C_Z, H), lambda i: (0, 0)),
        ],
        out_specs=pl.BlockSpec((512, H), lambda i: (i, 0)),
        out_shape=jax.ShapeDtypeStruct((B, H), jnp.float32),
    )


def _values_body(e_real, la_ref, gb_ref, r16_ref, s3_ref, p1_ref, p2_ref, o_ref):
    rowid = pl.program_id(0) * 512 + lax.broadcasted_iota(jnp.int32, (512, 1), 0)
    mask = (rowid < e_real).astype(jnp.float32)
    a = jnp.exp(la_ref[...]) * mask
    gb = gb_ref[...]
    o_ref[:, :128] = jnp.dot(a, r16_ref[...], preferred_element_type=jnp.float32) * gb[:, :C_S]
    avp = jnp.dot(a, s3_ref[...], preferred_element_type=jnp.float32) * gb[:, C_S:320]
    o_ref[:, 128:256] = avp[:, :128]
    o_ref[:, 256:384] = (jnp.dot(avp[:, 128:], p1_ref[...], preferred_element_type=jnp.float32)
                         + jnp.dot(a, p2_ref[...], preferred_element_type=jnp.float32))


@functools.lru_cache(maxsize=None)
def _make_values(B, e_real):
    return pl.pallas_call(
        functools.partial(_values_body, e_real),
        grid=(B // 512,),
        in_specs=[
            pl.BlockSpec((512, H), lambda i: (i, 0)),
            pl.BlockSpec((512, 384), lambda i: (i, 0)),
            pl.BlockSpec((H, 128), lambda i: (0, 0)),
            pl.BlockSpec((H, 192), lambda i: (0, 0)),
            pl.BlockSpec((64, 128), lambda i: (0, 0)),
            pl.BlockSpec((H, 128), lambda i: (0, 0)),
        ],
        out_specs=pl.BlockSpec((512, 384), lambda i: (i, 0)),
        out_shape=jax.ShapeDtypeStruct((B, 384), jnp.float32),
    )


_NPC = {}


def _np_consts():
    if 'S1' not in _NPC:
        S1 = np.zeros((C_S, H), np.float32)
        for h in range(H):
            S1[h * C_H:(h + 1) * C_H, h] = 1.0
        S2 = np.zeros((96, H), np.float32)
        for h in range(H):
            S2[h * 12:(h + 1) * 12, h] = 1.0
        R16 = np.zeros((H, 128), np.float32)
        for h in range(H):
            R16[h, h * C_H:(h + 1) * C_H] = 1.0
        S3 = np.zeros((H, 192), np.float32)
        for h in range(H):
            S3[h, h * 24:(h + 1) * 24] = 1.0
        P1 = np.zeros((64, 128), np.float32)
        P1[:64, :64] = np.eye(64, dtype=np.float32)
        P2 = np.zeros((H, 128), np.float32)
        for h in range(H):
            P2[h, 64 + h] = 1.0
        _NPC.update(S1=jnp.asarray(S1), S2=jnp.asarray(S2), R16=jnp.asarray(R16),
                    S3=jnp.asarray(S3), P1=jnp.asarray(P1), P2=jnp.asarray(P2))
    return _NPC


def _nt_body(x_ref, w1_ref, w2_ref, o_ref):
    h = jnp.maximum(jnp.dot(x_ref[...], w1_ref[...],
                            preferred_element_type=jnp.float32), 0.0)
    o_ref[...] = jnp.dot(h, w2_ref[...], preferred_element_type=jnp.float32)


def _nt_pallas(x, w1, w2):
    xp = jnp.zeros((N_PAD, C_S), jnp.float32).at[:N].set(x)
    out = pl.pallas_call(
        _nt_body,
        grid=(N_PAD // 512,),
        in_specs=[
            pl.BlockSpec((512, C_S), lambda i: (i, 0)),
            pl.BlockSpec((C_S, 2 * C_S), lambda i: (0, 0)),
            pl.BlockSpec((2 * C_S, C_S), lambda i: (0, 0)),
        ],
        out_specs=pl.BlockSpec((512, C_S), lambda i: (i, 0)),
        out_shape=jax.ShapeDtypeStruct((N_PAD, C_S), jnp.float32),
    )(xp, w1, w2)
    return out[:N]


def _quat_to_rot(q):
    q = q / (jnp.linalg.norm(q, axis=-1, keepdims=True) + 1e-8)
    w, x, y, z = q[..., 0], q[..., 1], q[..., 2], q[..., 3]
    R = jnp.stack([1 - 2 * (y * y + z * z), 2 * (x * y - w * z), 2 * (x * z + w * y),
                   2 * (x * y + w * z), 1 - 2 * (x * x + z * z), 2 * (y * z - w * x),
                   2 * (x * z - w * y), 2 * (y * z + w * x), 1 - 2 * (x * x + y * y)], axis=-1)
    return R.reshape(q.shape[:-1] + (3, 3))


def _ln(x):
    m = x.mean(-1, keepdims=True)
    v = ((x - m) ** 2).mean(-1, keepdims=True)
    return (x - m) / jnp.sqrt(v + 1e-5)


def _vln(v):
    n2 = (v ** 2).sum(-1).mean(-1, keepdims=True)
    return v / jnp.sqrt(n2 + 1e-5)[..., None]


def _rbf(d, n, d_max=20.0):
    mu = jnp.linspace(0.0, d_max, n)
    sigma = d_max / n
    return jnp.exp(-(((d[..., None] - mu) / sigma) ** 2))


def _posemb(diff, n):
    freq = jnp.exp(jnp.arange(0, n, 2, dtype=jnp.float32) * (-np.log(10000.0) / n))
    ang = diff[..., None].astype(jnp.float32) * freq
    return jnp.concatenate([jnp.cos(ang), jnp.sin(ang)], axis=-1)


def _pad_idx(idx, B):
    return jnp.concatenate([idx, jnp.zeros((B - idx.shape[0],), jnp.int32)])


def _attn_sc(p, s, rots, trans, z, src_pad, dst_pad, e_real, v_in):
    cst = _np_consts()
    vn = jnp.sqrt((v_in ** 2).sum(-1) + 1e-8)
    wcat = jnp.concatenate([p['Wq'], p['Wk'], p['Wv'], p['Wqp'], p['Wkp'], p['Wvp']], axis=1)
    x2 = jnp.concatenate([s, vn], axis=1)
    w2 = jnp.concatenate([wcat, p['Wvn'] @ wcat], axis=0)
    proj = _mm(x2, w2)
    q, k, v = proj[:, :C_S], proj[:, C_S:256], proj[:, 256:384]

    def to_global(flat, npts):
        pts = flat.reshape(N, H, npts, 3)
        return (jnp.einsum('nij,nhpj->nhpi', rots, pts)
                + trans[:, None, None, :]).reshape(N, H * npts * 3)

    qp = to_global(proj[:, 384:480], P_QK)
    kp = to_global(proj[:, 480:576], P_QK)
    vp = to_global(proj[:, 576:768], P_V)

    pad32 = jnp.zeros((N, 32), jnp.float32)
    dst_tab = jnp.concatenate([q, qp, pad32], axis=1)
    ga_tab = jnp.concatenate([k, kp, pad32], axis=1)
    gb_tab = jnp.concatenate([v, vp, pad32, pad32], axis=1)
    gd = _sc_gather(dst_tab, dst_pad)
    ga = _sc_gather(ga_tab, src_pad)
    gb = _sc_gather(gb_tab, src_pad)

    Bp = src_pad.shape[0]
    gamma = jax.nn.softplus(p['head_w'])
    s2g = cst['S2'] * gamma[None, :]
    logits = _make_logits(Bp)(gd, ga, z, cst['S1'], s2g, p['Wb'])
    m = jnp.max(logits)
    vals = _make_values(Bp, e_real)(logits - m, gb, cst['R16'], cst['S3'],
                                    cst['P1'], cst['P2'])
    s0, s1, s2 = _sc_scatter3(vals, dst_pad)

    den = s2[:, 64:72]
    deninv = 1.0 / jnp.maximum(den, 1e-30)
    o = s0.reshape(N, H, C_H) * deninv[:, :, None]
    op = (jnp.concatenate([s1, s2[:, :64]], axis=1).reshape(N, H, P_V * 3)
          * deninv[:, :, None]).reshape(N, H, P_V, 3)
    op_local = jnp.einsum('nji,nhpj->nhpi', rots, op - trans[:, None, None, :])
    opn = jnp.sqrt((op_local ** 2).sum(-1) + 1e-8)
    feat = jnp.concatenate([o.reshape(N, -1), op_local.reshape(N, -1),
                            opn.reshape(N, -1)], axis=-1)
    ds = _mm(feat, p['Wo_s'])
    dv = _mm(op_local.reshape(N, H * P_V * 3),
             jnp.kron(p['Wo_v'], jnp.eye(3, dtype=jnp.float32)))
    return ds, dv.reshape(N, C_V, 3)


def kernel(quats, trans, t, noising_mask, x_mask, edge_index, seq_edge_index, params):
    src = _pad_idx(edge_index[0], E_P)
    dst = _pad_idx(edge_index[1], E_P)
    ssrc = _pad_idx(seq_edge_index[0], ES_P)
    sdst = _pad_idx(seq_edge_index[1], ES_P)
    all_idx = jnp.concatenate([src, dst, ssrc, sdst])

    rots = _quat_to_rot(quats)
    keep = (~x_mask).astype(jnp.float32)
    center = trans.mean(axis=0, keepdims=True)
    trans = (trans - center) * 0.1

    trans128 = jnp.zeros((N, 128), jnp.float32).at[:, :3].set(trans)
    gt = _sc_gather(trans128, all_idx)[:, :3]
    gt_src, gt_dst = gt[:E_P], gt[E_P:2 * E_P]
    gt_ssrc, gt_sdst = gt[2 * E_P:2 * E_P + ES_P], gt[2 * E_P + ES_P:]

    def efeats(gs, gdd, spad, dpad):
        d = jnp.sqrt(((gs - gdd) ** 2).sum(-1) + 1e-12)
        return jnp.concatenate([_rbf(d, C_Z // 2), _posemb(spad - dpad, C_Z // 2)],
                               axis=-1)

    z = efeats(gt_src, gt_dst, src, dst)
    zs = efeats(gt_ssrc, gt_sdst, ssrc, sdst)

    ft = _rbf(t, H_TIME, 1.0)
    et = jax.nn.relu(_mm(jax.nn.relu(_mm(ft, params['Wt1']) + params['bt1']),
                         params['Wt2']) + params['bt2'])
    res_pos = _posemb(jnp.arange(N), C_S)
    node = _mm(jnp.concatenate([res_pos, et, noising_mask[:, None]], axis=-1),
               params['Wemb']) + params['bemb']
    vecs = jnp.zeros((N, C_V, 3), jnp.float32)

    for lp in params['layers']:
        gn = _sc_gather(node, all_idx)
        z = _edge_tr_sc(lp['edge_tr'], z, gn[:E_P], gn[E_P:2 * E_P])
        zs = _edge_tr_sc(lp['seq_edge_tr'], zs, gn[2 * E_P:2 * E_P + ES_P],
                         gn[2 * E_P + ES_P:])
        ds, dv = _attn_sc(lp['attn_seq'], node, rots, trans, zs, ssrc, sdst,
                          E_SEQ, vecs)
        node = _ln(node + ds * keep[:, None])
        vecs = _vln(vecs + dv * keep[:, None, None])
        ds, dv = _attn_sc(lp['attn_spatial'], node, rots, trans, z, src, dst,
                          E, vecs)
        node = _ln(node + ds * keep[:, None])
        vecs = _vln(vecs + dv * keep[:, None, None])
        vn = jnp.sqrt((vecs ** 2).sum(-1) + 1e-8)
        h = _mm(jnp.concatenate([node, vn], axis=-1), lp['lfu_W1'], act='relu')
        ds = _mm(h, lp['lfu_W2'])
        gate = _mm(node, lp['lfu_Wg'], act='sigmoid')
        dv = gate[..., None] * _mm(vecs.reshape(N, 3 * C_V),
                                   jnp.kron(lp['lfu_Wvm'],
                                            jnp.eye(3, dtype=jnp.float32))
                                   ).reshape(N, C_V, 3)
        node = _ln(node + ds * keep[:, None])
        vecs = _vln(vecs + dv * keep[:, None, None])
        node = _ln(node + _nt_pallas(node, lp['nt_W1'], lp['nt_W2']))
        vecs = vecs * jax.nn.sigmoid(_mm(node, lp['nt_Wg']))[..., None]
        node = node * keep[:, None]
        vecs = vecs * keep[:, None, None]
        u = (_mm(node * noising_mask[:, None], lp['bb_Ws'])
             + _mm((vecs * noising_mask[:, None, None]).reshape(N, -1), lp['bb_Wv']))
        u = u * noising_mask[:, None]
        new_trans = trans + jnp.einsum('nij,nj->ni', rots, u[:, 3:])
        Ru = _quat_to_rot(jnp.concatenate([jnp.ones((N, 1), jnp.float32), u[:, :3]], axis=-1))
        rots = jnp.einsum('nij,njk->nik', rots, Ru)
        trans = new_trans
    trans = trans * 10.0 + center
    return node, trans, rots, vecs
